# baseline (speedup 1.0000x reference)
import functools

import jax
import jax.numpy as jnp
import numpy as np
from jax import lax
from jax.experimental import pallas as pl
from jax.experimental.pallas import tpu as pltpu
from jax.experimental.pallas import tpu_sc as plsc

NSPH = 7
NRAD = 6
CUT = 5.0
ENV_P = 6


def _sph_jn_np(x, n):
    x = np.asarray(x, dtype=np.float64)
    j0 = np.sin(x) / x
    if n == 0:
        return j0
    j1 = np.sin(x) / x ** 2 - np.cos(x) / x
    jm1, jc = j0, j1
    for l in range(1, n):
        jm1, jc = jc, (2 * l + 1) / x * jc - jm1
    return jc


def _sph_jn_zeros(n, k):
    zerosj = np.zeros((n, k), dtype=np.float64)
    zerosj[0] = np.arange(1, k + 1) * np.pi
    points = np.arange(1, k + n) * np.pi
    for i in range(1, n):
        m = k + n - 1 - i
        racines = np.zeros(m)
        for j in range(m):
            a, b = points[j], points[j + 1]
            fa = _sph_jn_np(a, i)
            for _ in range(100):
                c = 0.5 * (a + b)
                fc = _sph_jn_np(c, i)
                if fa * fc <= 0:
                    b = c
                else:
                    a, fa = c, fc
            racines[j] = 0.5 * (a + b)
        points = racines
        zerosj[i, :k] = racines[:k]
    return zerosj


_BZ = _sph_jn_zeros(NSPH, NRAD)
_BN = np.zeros((NSPH, NRAD))
for _o in range(NSPH):
    _BN[_o] = 1.0 / np.sqrt(0.5 * _sph_jn_np(_BZ[_o], _o + 1) ** 2)

_NF = NSPH * NRAD
_LANES = 48

_ZF = np.ones((1, _LANES), np.float32)
_ZF[0, :_NF] = _BZ.reshape(-1).astype(np.float32)
_NRMF = np.zeros((1, _LANES), np.float32)
_NRMF[0, :_NF] = _BN.reshape(-1).astype(np.float32)
_ORDF = np.zeros((1, _LANES), np.float32)
_ORDF[0, :_NF] = np.repeat(np.arange(NSPH), NRAD).astype(np.float32)
_SPHC = [np.sqrt((2 * l + 1) / (4 * np.pi)).astype(np.float32) for l in range(NSPH)]

_NC = 2
_NS = 16
_NW = _NC * _NS
_CHUNK = 128
_EDIM = 16


def _edge_body(d_ref, e_ref):
    v = d_ref[...]
    n2 = jnp.sum(v * v, axis=1, keepdims=True)
    dist = jnp.sqrt(n2)
    nd = v / dist
    dsc = dist / CUT
    pad = jnp.zeros((v.shape[0], _EDIM - 4), jnp.float32)
    e_ref[...] = jnp.concatenate([nd, dsc, pad], axis=1)


def _edge_table(distances, bn=2000):
    n = distances.shape[0]
    return pl.pallas_call(
        _edge_body,
        grid=(n // bn,),
        in_specs=[pl.BlockSpec((bn, 3), lambda i: (i, 0))],
        out_specs=pl.BlockSpec((bn, _EDIM), lambda i: (i, 0)),
        out_shape=jax.ShapeDtypeStruct((n, _EDIM), jnp.float32),
    )(distances)


def _sc_body(cpw, e_hbm, s_hbm, r_hbm, c_hbm, sidx, ridx, srow, rrow, outb, sem):
    wid = lax.axis_index("s") * _NC + lax.axis_index("c")

    def step(j, carry):
        base = (wid * cpw + j) * _CHUNK
        pltpu.sync_copy(s_hbm.at[pl.ds(base, _CHUNK)], sidx)
        pltpu.sync_copy(r_hbm.at[pl.ds(base, _CHUNK)], ridx)
        cp1 = pltpu.async_copy(e_hbm.at[sidx], srow, sem)
        cp2 = pltpu.async_copy(e_hbm.at[ridx], rrow, sem)
        cp1.wait()
        cp2.wait()
        for g in range(_CHUNK // 16):
            t = lax.iota(jnp.int32, 16) + g * 16

            def col(rowref, c):
                return plsc.load_gather(rowref, [t, jnp.full((16,), c, jnp.int32)])

            sx, sy, sz, sd = col(srow, 0), col(srow, 1), col(srow, 2), col(srow, 3)
            rx, ry, rz = col(rrow, 0), col(rrow, 1), col(rrow, 2)
            cosang = sx * rx + sy * ry + sz * rz
            plsc.store_scatter(outb, [t, jnp.zeros((16,), jnp.int32)], cosang)
            plsc.store_scatter(outb, [t, jnp.ones((16,), jnp.int32)], sd)
        pltpu.sync_copy(outb, c_hbm.at[pl.ds(base, _CHUNK)])
        return carry

    lax.fori_loop(0, cpw, step, 0)


def _sc_gather(etab, spad, rpad):
    tpad = spad.shape[0]
    cpw = tpad // (_NW * _CHUNK)
    mesh = plsc.VectorSubcoreMesh(
        core_axis_name="c", subcore_axis_name="s", num_cores=_NC, num_subcores=_NS
    )
    kfn = pl.kernel(
        functools.partial(_sc_body, cpw),
        out_type=jax.ShapeDtypeStruct((tpad, 2), jnp.float32),
        mesh=mesh,
        compiler_params=pltpu.CompilerParams(
            use_tc_tiling_on_sc=False, needs_layout_passes=False
        ),
        scratch_types=[
            pltpu.VMEM((_CHUNK,), jnp.int32),
            pltpu.VMEM((_CHUNK,), jnp.int32),
            pltpu.VMEM((_CHUNK, _EDIM), jnp.float32),
            pltpu.VMEM((_CHUNK, _EDIM), jnp.float32),
            pltpu.VMEM((_CHUNK, 2), jnp.float32),
            pltpu.SemaphoreType.DMA,
        ],
    )
    return kfn(etab, spad, rpad)


_CONSTS = np.zeros((8, _LANES), np.float32)
_CONSTS[0] = _ZF[0]
_CONSTS[1] = _ORDF[0]
_CONSTS[2] = _NRMF[0]


def _basis_body(k_ref, c_ref, o_ref):
    cb = c_ref[...]
    cosang = cb[:, 0:1]
    d = cb[:, 1:2]
    zf = k_ref[0:1, :]
    ordf = k_ref[1:2, :]
    nrmf = k_ref[2:3, :]

    x = d * zf
    sx = jnp.sin(x)
    cx = jnp.cos(x)
    j0 = sx / x
    j1 = sx / (x * x) - cx / x
    jsel = jnp.where(ordf >= 1.0, j1, j0)
    jm1, jc = j0, j1
    for l in range(1, NSPH - 1):
        jn = (2 * l + 1) / x * jc - jm1
        jm1, jc = jc, jn
        jsel = jnp.where(ordf >= float(l + 1), jn, jsel)

    p_prev = jnp.ones_like(cosang)
    p_cur = cosang
    csel = jnp.where(ordf >= 1.0, _SPHC[1] * p_cur, _SPHC[0] * p_prev)
    for l in range(1, NSPH - 1):
        p_next = ((2 * l + 1) * cosang * p_cur - l * p_prev) / (l + 1)
        p_prev, p_cur = p_cur, p_next
        csel = jnp.where(ordf >= float(l + 1), _SPHC[l + 1] * p_cur, csel)

    a = -(ENV_P + 1) * (ENV_P + 2) / 2.0
    b = float(ENV_P * (ENV_P + 2))
    c = -ENV_P * (ENV_P + 1) / 2.0
    env = 1.0 / d + a * d ** (ENV_P - 1) + b * d ** ENV_P + c * d ** (ENV_P + 1)
    env = jnp.where(d < 1.0, env, jnp.zeros_like(env))

    out = jsel * nrmf * csel * env
    o_ref[...] = out[:, :_NF]


def _basis(cstream, t, bs=1600):
    return pl.pallas_call(
        _basis_body,
        grid=(t // bs,),
        in_specs=[
            pl.BlockSpec((8, _LANES), lambda i: (0, 0)),
            pl.BlockSpec((bs, 2), lambda i: (i, 0)),
        ],
        out_specs=pl.BlockSpec((bs, _NF), lambda i: (i, 0)),
        out_shape=jax.ShapeDtypeStruct((t, _NF), jnp.float32),
    )(jnp.asarray(_CONSTS), cstream)


def kernel(distances, senders, receivers):
    t = senders.shape[0]
    superchunk = _NW * _CHUNK
    tpad = ((t + superchunk - 1) // superchunk) * superchunk
    spad = jnp.concatenate([senders.astype(jnp.int32), jnp.zeros((tpad - t,), jnp.int32)])
    rpad = jnp.concatenate([receivers.astype(jnp.int32), jnp.zeros((tpad - t,), jnp.int32)])
    etab = _edge_table(distances)
    cstream = _sc_gather(etab, spad, rpad)
    return _basis(cstream, t)


# speedup vs baseline: 1.0584x; 1.0584x over previous
import functools

import jax
import jax.numpy as jnp
import numpy as np
from jax import lax
from jax.experimental import pallas as pl
from jax.experimental.pallas import tpu as pltpu
from jax.experimental.pallas import tpu_sc as plsc

NSPH = 7
NRAD = 6
CUT = 5.0
ENV_P = 6


def _sph_jn_np(x, n):
    x = np.asarray(x, dtype=np.float64)
    j0 = np.sin(x) / x
    if n == 0:
        return j0
    j1 = np.sin(x) / x ** 2 - np.cos(x) / x
    jm1, jc = j0, j1
    for l in range(1, n):
        jm1, jc = jc, (2 * l + 1) / x * jc - jm1
    return jc


def _sph_jn_zeros(n, k):
    zerosj = np.zeros((n, k), dtype=np.float64)
    zerosj[0] = np.arange(1, k + 1) * np.pi
    points = np.arange(1, k + n) * np.pi
    for i in range(1, n):
        m = k + n - 1 - i
        racines = np.zeros(m)
        for j in range(m):
            a, b = points[j], points[j + 1]
            fa = _sph_jn_np(a, i)
            for _ in range(100):
                c = 0.5 * (a + b)
                fc = _sph_jn_np(c, i)
                if fa * fc <= 0:
                    b = c
                else:
                    a, fa = c, fc
            racines[j] = 0.5 * (a + b)
        points = racines
        zerosj[i, :k] = racines[:k]
    return zerosj


_BZ = _sph_jn_zeros(NSPH, NRAD)
_BN = np.zeros((NSPH, NRAD))
for _o in range(NSPH):
    _BN[_o] = 1.0 / np.sqrt(0.5 * _sph_jn_np(_BZ[_o], _o + 1) ** 2)

_NF = NSPH * NRAD
_LANES = 48

_ZF = np.ones((1, _LANES), np.float32)
_ZF[0, :_NF] = _BZ.reshape(-1).astype(np.float32)
_NRMF = np.zeros((1, _LANES), np.float32)
_NRMF[0, :_NF] = _BN.reshape(-1).astype(np.float32)
_ORDF = np.zeros((1, _LANES), np.float32)
_ORDF[0, :_NF] = np.repeat(np.arange(NSPH), NRAD).astype(np.float32)
_SPHC = [np.sqrt((2 * l + 1) / (4 * np.pi)).astype(np.float32) for l in range(NSPH)]

_NC = 2
_NS = 16
_NW = _NC * _NS
_CHUNK = 128
_EDIM = 16


def _edge_body(d_ref, n_ref, e_ref):
    v = d_ref[...]
    dist = n_ref[...]
    nd = v / dist
    dsc = dist / CUT
    pad = jnp.zeros((v.shape[0], _EDIM - 4), jnp.float32)
    e_ref[...] = jnp.concatenate([nd, dsc, pad], axis=1)


def _edge_table(distances, dist, bn=2000):
    n = distances.shape[0]
    return pl.pallas_call(
        _edge_body,
        grid=(n // bn,),
        in_specs=[
            pl.BlockSpec((bn, 3), lambda i: (i, 0)),
            pl.BlockSpec((bn, 1), lambda i: (i, 0)),
        ],
        out_specs=pl.BlockSpec((bn, _EDIM), lambda i: (i, 0)),
        out_shape=jax.ShapeDtypeStruct((n, _EDIM), jnp.float32),
    )(distances, dist)


def _sc_body(cpw, e_hbm, s_hbm, r_hbm, c_hbm, sidx, ridx, srow, rrow, outb, sem):
    wid = lax.axis_index("s") * _NC + lax.axis_index("c")

    def step(j, carry):
        base = (wid * cpw + j) * _CHUNK
        pltpu.sync_copy(s_hbm.at[pl.ds(base, _CHUNK)], sidx)
        pltpu.sync_copy(r_hbm.at[pl.ds(base, _CHUNK)], ridx)
        cp1 = pltpu.async_copy(e_hbm.at[sidx], srow, sem)
        cp2 = pltpu.async_copy(e_hbm.at[ridx], rrow, sem)
        cp1.wait()
        cp2.wait()
        for g in range(_CHUNK // 16):
            t = lax.iota(jnp.int32, 16) + g * 16

            def col(rowref, c):
                return plsc.load_gather(rowref, [t, jnp.full((16,), c, jnp.int32)])

            sx, sy, sz, sd = col(srow, 0), col(srow, 1), col(srow, 2), col(srow, 3)
            rx, ry, rz = col(rrow, 0), col(rrow, 1), col(rrow, 2)
            cosang = sx * rx + sy * ry + sz * rz
            plsc.store_scatter(outb, [t, jnp.zeros((16,), jnp.int32)], cosang)
            plsc.store_scatter(outb, [t, jnp.ones((16,), jnp.int32)], sd)
        pltpu.sync_copy(outb, c_hbm.at[pl.ds(base, _CHUNK)])
        return carry

    lax.fori_loop(0, cpw, step, 0)


def _sc_gather(etab, spad, rpad):
    tpad = spad.shape[0]
    cpw = tpad // (_NW * _CHUNK)
    mesh = plsc.VectorSubcoreMesh(
        core_axis_name="c", subcore_axis_name="s", num_cores=_NC, num_subcores=_NS
    )
    kfn = pl.kernel(
        functools.partial(_sc_body, cpw),
        out_type=jax.ShapeDtypeStruct((tpad, 2), jnp.float32),
        mesh=mesh,
        compiler_params=pltpu.CompilerParams(
            use_tc_tiling_on_sc=False, needs_layout_passes=False
        ),
        scratch_types=[
            pltpu.VMEM((_CHUNK,), jnp.int32),
            pltpu.VMEM((_CHUNK,), jnp.int32),
            pltpu.VMEM((_CHUNK, _EDIM), jnp.float32),
            pltpu.VMEM((_CHUNK, _EDIM), jnp.float32),
            pltpu.VMEM((_CHUNK, 2), jnp.float32),
            pltpu.SemaphoreType.DMA,
        ],
    )
    return kfn(etab, spad, rpad)


_PK = 2
_PLANES = _PK * _NF
_CONSTS = np.zeros((8, _PLANES), np.float32)
_CONSTS[0] = np.tile(_ZF[0, :_NF], _PK)
_CONSTS[1] = np.tile(_ORDF[0, :_NF], _PK)
_CONSTS[2] = np.tile(_NRMF[0, :_NF], _PK)


def _basis_body(k_ref, c_ref, o_ref):
    cb = c_ref[...]
    bs = cb.shape[0]
    cosang = jnp.concatenate(
        [
            lax.broadcast_in_dim(cb[:, 2 * p : 2 * p + 1], (bs, _NF), (0, 1))
            for p in range(_PK)
        ],
        axis=1,
    )
    d = jnp.concatenate(
        [
            lax.broadcast_in_dim(cb[:, 2 * p + 1 : 2 * p + 2], (bs, _NF), (0, 1))
            for p in range(_PK)
        ],
        axis=1,
    )
    zf = k_ref[0:1, :]
    ordf = k_ref[1:2, :]
    nrmf = k_ref[2:3, :]

    x = d * zf
    sx = jnp.sin(x)
    cx = jnp.cos(x)
    j0 = sx / x
    j1 = sx / (x * x) - cx / x
    jsel = jnp.where(ordf >= 1.0, j1, j0)
    jm1, jc = j0, j1
    for l in range(1, NSPH - 1):
        jn = (2 * l + 1) / x * jc - jm1
        jm1, jc = jc, jn
        jsel = jnp.where(ordf >= float(l + 1), jn, jsel)

    p_prev = jnp.ones_like(cosang)
    p_cur = cosang
    csel = jnp.where(ordf >= 1.0, _SPHC[1] * p_cur, _SPHC[0] * p_prev)
    for l in range(1, NSPH - 1):
        p_next = ((2 * l + 1) * cosang * p_cur - l * p_prev) * (1.0 / (l + 1))
        p_prev, p_cur = p_cur, p_next
        csel = jnp.where(ordf >= float(l + 1), _SPHC[l + 1] * p_cur, csel)

    a = -(ENV_P + 1) * (ENV_P + 2) / 2.0
    b = float(ENV_P * (ENV_P + 2))
    c = -ENV_P * (ENV_P + 1) / 2.0
    env = 1.0 / d + a * d ** (ENV_P - 1) + b * d ** ENV_P + c * d ** (ENV_P + 1)
    env = jnp.where(d < 1.0, env, jnp.zeros_like(env))

    out = jsel * nrmf * csel * env
    o_ref[...] = out


def _basis(cstream, t, bs=1600):
    rows = t // _PK
    cpk = cstream.reshape(-1, 2 * _PK)
    out = pl.pallas_call(
        _basis_body,
        grid=(rows // bs,),
        in_specs=[
            pl.BlockSpec((8, _PLANES), lambda i: (0, 0)),
            pl.BlockSpec((bs, 2 * _PK), lambda i: (i, 0)),
        ],
        out_specs=pl.BlockSpec((bs, _PLANES), lambda i: (i, 0)),
        out_shape=jax.ShapeDtypeStruct((rows, _PLANES), jnp.float32),
    )(jnp.asarray(_CONSTS), cpk)
    return out.reshape(t, _NF)


def kernel(distances, senders, receivers):
    t = senders.shape[0]
    superchunk = _NW * _CHUNK
    tpad = ((t + superchunk - 1) // superchunk) * superchunk
    spad = jnp.concatenate([senders.astype(jnp.int32), jnp.zeros((tpad - t,), jnp.int32)])
    rpad = jnp.concatenate([receivers.astype(jnp.int32), jnp.zeros((tpad - t,), jnp.int32)])
    dist = jnp.linalg.norm(distances, axis=-1, keepdims=True)
    etab = _edge_table(distances, dist)
    cstream = _sc_gather(etab, spad, rpad)
    return _basis(cstream, t)


# speedup vs baseline: 1.2127x; 1.1458x over previous
import functools

import jax
import jax.numpy as jnp
import numpy as np
from jax import lax
from jax.experimental import pallas as pl
from jax.experimental.pallas import tpu as pltpu
from jax.experimental.pallas import tpu_sc as plsc

NSPH = 7
NRAD = 6
CUT = 5.0
ENV_P = 6


def _sph_jn_np(x, n):
    x = np.asarray(x, dtype=np.float64)
    j0 = np.sin(x) / x
    if n == 0:
        return j0
    j1 = np.sin(x) / x ** 2 - np.cos(x) / x
    jm1, jc = j0, j1
    for l in range(1, n):
        jm1, jc = jc, (2 * l + 1) / x * jc - jm1
    return jc


def _sph_jn_zeros(n, k):
    zerosj = np.zeros((n, k), dtype=np.float64)
    zerosj[0] = np.arange(1, k + 1) * np.pi
    points = np.arange(1, k + n) * np.pi
    for i in range(1, n):
        m = k + n - 1 - i
        racines = np.zeros(m)
        for j in range(m):
            a, b = points[j], points[j + 1]
            fa = _sph_jn_np(a, i)
            for _ in range(100):
                c = 0.5 * (a + b)
                fc = _sph_jn_np(c, i)
                if fa * fc <= 0:
                    b = c
                else:
                    a, fa = c, fc
            racines[j] = 0.5 * (a + b)
        points = racines
        zerosj[i, :k] = racines[:k]
    return zerosj


_BZ = _sph_jn_zeros(NSPH, NRAD)
_BN = np.zeros((NSPH, NRAD))
for _o in range(NSPH):
    _BN[_o] = 1.0 / np.sqrt(0.5 * _sph_jn_np(_BZ[_o], _o + 1) ** 2)

_NF = NSPH * NRAD
_LANES = 48

_ZF = np.ones((1, _LANES), np.float32)
_ZF[0, :_NF] = _BZ.reshape(-1).astype(np.float32)
_NRMF = np.zeros((1, _LANES), np.float32)
_NRMF[0, :_NF] = _BN.reshape(-1).astype(np.float32)
_ORDF = np.zeros((1, _LANES), np.float32)
_ORDF[0, :_NF] = np.repeat(np.arange(NSPH), NRAD).astype(np.float32)
_SPHC = [np.sqrt((2 * l + 1) / (4 * np.pi)).astype(np.float32) for l in range(NSPH)]

_NC = 2
_NS = 16
_NW = _NC * _NS
_CHUNK = 128
_EDIM = 8


def _edge_body(d_ref, n_ref, e_ref):
    v = d_ref[...]
    dist = n_ref[...]
    nd = v / dist
    dsc = dist / CUT
    pad = jnp.zeros((v.shape[0], _EDIM - 4), jnp.float32)
    e_ref[...] = jnp.concatenate([nd, dsc, pad], axis=1)


def _edge_table(distances, dist, bn=2000):
    n = distances.shape[0]
    return pl.pallas_call(
        _edge_body,
        grid=(n // bn,),
        in_specs=[
            pl.BlockSpec((bn, 3), lambda i: (i, 0)),
            pl.BlockSpec((bn, 1), lambda i: (i, 0)),
        ],
        out_specs=pl.BlockSpec((bn, _EDIM), lambda i: (i, 0)),
        out_shape=jax.ShapeDtypeStruct((n, _EDIM), jnp.float32),
    )(distances, dist)


def _sc_body(tpw, e_hbm, s_hbm, r_hbm, c_hbm, sidx, ridx, srow, rrow, outb, sem):
    wid = lax.axis_index("s") * _NC + lax.axis_index("c")
    wbase = wid * tpw
    ncp = (tpw + _CHUNK - 1) // _CHUNK
    last_off = tpw - _CHUNK

    def step(j, carry):
        base = wbase + jnp.minimum(j * _CHUNK, last_off)
        cp_s = pltpu.async_copy(s_hbm.at[pl.ds(base, _CHUNK)], sidx, sem)
        cp_r = pltpu.async_copy(r_hbm.at[pl.ds(base, _CHUNK)], ridx, sem)
        cp_s.wait()
        cp_r.wait()
        cp1 = pltpu.async_copy(e_hbm.at[sidx], srow, sem)
        cp2 = pltpu.async_copy(e_hbm.at[ridx], rrow, sem)
        cp1.wait()
        cp2.wait()
        for g in range(_CHUNK // 16):
            t = lax.iota(jnp.int32, 16) + g * 16

            def col(rowref, c):
                return plsc.load_gather(rowref, [t, jnp.full((16,), c, jnp.int32)])

            sx, sy, sz, sd = col(srow, 0), col(srow, 1), col(srow, 2), col(srow, 3)
            rx, ry, rz = col(rrow, 0), col(rrow, 1), col(rrow, 2)
            cosang = sx * rx + sy * ry + sz * rz
            row = lax.shift_right_logical(t, 1)
            colb = lax.mul(lax.bitwise_and(t, 1), 2)
            plsc.store_scatter(outb, [row, colb], cosang)
            plsc.store_scatter(outb, [row, colb + 1], sd)
        pltpu.sync_copy(outb, c_hbm.at[pl.ds(lax.shift_right_logical(base, 1), _CHUNK // 2)])
        return carry

    lax.fori_loop(0, ncp, step, 0)


def _sc_gather(etab, senders, receivers):
    t = senders.shape[0]
    tpw = t // _NW
    mesh = plsc.VectorSubcoreMesh(
        core_axis_name="c", subcore_axis_name="s", num_cores=_NC, num_subcores=_NS
    )
    kfn = pl.kernel(
        functools.partial(_sc_body, tpw),
        out_type=jax.ShapeDtypeStruct((t // 2, 4), jnp.float32),
        mesh=mesh,
        compiler_params=pltpu.CompilerParams(
            use_tc_tiling_on_sc=False, needs_layout_passes=False
        ),
        scratch_types=[
            pltpu.VMEM((_CHUNK,), jnp.int32),
            pltpu.VMEM((_CHUNK,), jnp.int32),
            pltpu.VMEM((_CHUNK, _EDIM), jnp.float32),
            pltpu.VMEM((_CHUNK, _EDIM), jnp.float32),
            pltpu.VMEM((_CHUNK // 2, 4), jnp.float32),
            pltpu.SemaphoreType.DMA,
        ],
    )
    return kfn(etab, senders, receivers)


_PK = 2
_PLANES = _PK * _NF
_CONSTS = np.zeros((8, _PLANES), np.float32)
_CONSTS[0] = np.tile(_ZF[0, :_NF], _PK)
_CONSTS[1] = np.tile(_ORDF[0, :_NF], _PK)
_CONSTS[2] = np.tile(_NRMF[0, :_NF], _PK)


def _basis_body(k_ref, c_ref, o_ref):
    cb = c_ref[...]
    bs = cb.shape[0]
    cosang = jnp.concatenate(
        [
            lax.broadcast_in_dim(cb[:, 2 * p : 2 * p + 1], (bs, _NF), (0, 1))
            for p in range(_PK)
        ],
        axis=1,
    )
    d = jnp.concatenate(
        [
            lax.broadcast_in_dim(cb[:, 2 * p + 1 : 2 * p + 2], (bs, _NF), (0, 1))
            for p in range(_PK)
        ],
        axis=1,
    )
    zf = k_ref[0:1, :]
    ordf = k_ref[1:2, :]
    nrmf = k_ref[2:3, :]

    x = d * zf
    sx = jnp.sin(x)
    cx = jnp.cos(x)
    j0 = sx / x
    j1 = sx / (x * x) - cx / x
    jsel = jnp.where(ordf >= 1.0, j1, j0)
    jm1, jc = j0, j1
    for l in range(1, NSPH - 1):
        jn = (2 * l + 1) / x * jc - jm1
        jm1, jc = jc, jn
        jsel = jnp.where(ordf >= float(l + 1), jn, jsel)

    p_prev = jnp.ones_like(cosang)
    p_cur = cosang
    csel = jnp.where(ordf >= 1.0, _SPHC[1] * p_cur, _SPHC[0] * p_prev)
    for l in range(1, NSPH - 1):
        p_next = ((2 * l + 1) * cosang * p_cur - l * p_prev) * (1.0 / (l + 1))
        p_prev, p_cur = p_cur, p_next
        csel = jnp.where(ordf >= float(l + 1), _SPHC[l + 1] * p_cur, csel)

    a = -(ENV_P + 1) * (ENV_P + 2) / 2.0
    b = float(ENV_P * (ENV_P + 2))
    c = -ENV_P * (ENV_P + 1) / 2.0
    env = 1.0 / d + a * d ** (ENV_P - 1) + b * d ** ENV_P + c * d ** (ENV_P + 1)
    env = jnp.where(d < 1.0, env, jnp.zeros_like(env))

    out = jsel * nrmf * csel * env
    o_ref[...] = out


def _basis(cstream, t, bs=1600):
    rows = t // _PK
    cpk = cstream
    out = pl.pallas_call(
        _basis_body,
        grid=(rows // bs,),
        in_specs=[
            pl.BlockSpec((8, _PLANES), lambda i: (0, 0)),
            pl.BlockSpec((bs, 2 * _PK), lambda i: (i, 0)),
        ],
        out_specs=pl.BlockSpec((bs, _PLANES), lambda i: (i, 0)),
        out_shape=jax.ShapeDtypeStruct((rows, _PLANES), jnp.float32),
    )(jnp.asarray(_CONSTS), cpk)
    return out.reshape(t, _NF)


def kernel(distances, senders, receivers):
    t = senders.shape[0]
    dist = jnp.linalg.norm(distances, axis=-1, keepdims=True)
    etab = _edge_table(distances, dist)
    cstream = _sc_gather(etab, senders.astype(jnp.int32), receivers.astype(jnp.int32))
    return _basis(cstream, t)


# speedup vs baseline: 1.8263x; 1.5060x over previous
import functools

import jax
import jax.numpy as jnp
import numpy as np
from jax import lax
from jax.experimental import pallas as pl
from jax.experimental.pallas import tpu as pltpu
from jax.experimental.pallas import tpu_sc as plsc

NSPH = 7
NRAD = 6
CUT = 5.0
ENV_P = 6


def _sph_jn_np(x, n):
    x = np.asarray(x, dtype=np.float64)
    j0 = np.sin(x) / x
    if n == 0:
        return j0
    j1 = np.sin(x) / x ** 2 - np.cos(x) / x
    jm1, jc = j0, j1
    for l in range(1, n):
        jm1, jc = jc, (2 * l + 1) / x * jc - jm1
    return jc


def _sph_jn_zeros(n, k):
    zerosj = np.zeros((n, k), dtype=np.float64)
    zerosj[0] = np.arange(1, k + 1) * np.pi
    points = np.arange(1, k + n) * np.pi
    for i in range(1, n):
        m = k + n - 1 - i
        racines = np.zeros(m)
        for j in range(m):
            a, b = points[j], points[j + 1]
            fa = _sph_jn_np(a, i)
            for _ in range(100):
                c = 0.5 * (a + b)
                fc = _sph_jn_np(c, i)
                if fa * fc <= 0:
                    b = c
                else:
                    a, fa = c, fc
            racines[j] = 0.5 * (a + b)
        points = racines
        zerosj[i, :k] = racines[:k]
    return zerosj


_BZ = _sph_jn_zeros(NSPH, NRAD)
_BN = np.zeros((NSPH, NRAD))
for _o in range(NSPH):
    _BN[_o] = 1.0 / np.sqrt(0.5 * _sph_jn_np(_BZ[_o], _o + 1) ** 2)

_NF = NSPH * NRAD
_LANES = 48

_ZF = np.ones((1, _LANES), np.float32)
_ZF[0, :_NF] = _BZ.reshape(-1).astype(np.float32)
_NRMF = np.zeros((1, _LANES), np.float32)
_NRMF[0, :_NF] = _BN.reshape(-1).astype(np.float32)
_ORDF = np.zeros((1, _LANES), np.float32)
_ORDF[0, :_NF] = np.repeat(np.arange(NSPH), NRAD).astype(np.float32)
_SPHC = [np.sqrt((2 * l + 1) / (4 * np.pi)).astype(np.float32) for l in range(NSPH)]

_NC = 2
_NS = 16
_NW = _NC * _NS
_CHUNK = 128
_EDIM = 8


_EC = 1024


def _sc_edge_body(npw, xs, ys, zs, ds, e_hbm, xb, yb, zb, db, eb, sem):
    wid = lax.axis_index("s") * _NC + lax.axis_index("c")
    wbase = wid * npw
    ncp = (npw + _EC - 1) // _EC
    last_off = npw - _EC

    def step(j, carry):
        base = wbase + jnp.minimum(j * _EC, last_off)
        cps = [
            pltpu.async_copy(src.at[pl.ds(base, _EC)], dst, sem)
            for src, dst in ((xs, xb), (ys, yb), (zs, zb), (ds, db))
        ]
        for cp in cps:
            cp.wait()
        for g in range(_EC // 16):
            t = lax.iota(jnp.int32, 16) + g * 16
            sl = pl.ds(g * 16, 16)
            dv = db[sl]
            nx = xb[sl] / dv
            ny = yb[sl] / dv
            nz = zb[sl] / dv
            dd = dv / CUT
            for c, val in ((0, nx), (1, ny), (2, nz), (3, dd)):
                plsc.store_scatter(eb, [t, jnp.full((16,), c, jnp.int32)], val)
        pltpu.sync_copy(eb, e_hbm.at[pl.ds(base, _EC)])
        return carry

    lax.fori_loop(0, ncp, step, 0)


def _edge_table(distances, dist1d):
    n = distances.shape[0]
    npw = n // _NW
    xs = distances[:, 0]
    ys = distances[:, 1]
    zs = distances[:, 2]
    mesh = plsc.VectorSubcoreMesh(
        core_axis_name="c", subcore_axis_name="s", num_cores=_NC, num_subcores=_NS
    )
    kfn = pl.kernel(
        functools.partial(_sc_edge_body, npw),
        out_type=jax.ShapeDtypeStruct((n, _EDIM), jnp.float32),
        mesh=mesh,
        compiler_params=pltpu.CompilerParams(
            use_tc_tiling_on_sc=False, needs_layout_passes=False
        ),
        scratch_types=[
            pltpu.VMEM((_EC,), jnp.float32),
            pltpu.VMEM((_EC,), jnp.float32),
            pltpu.VMEM((_EC,), jnp.float32),
            pltpu.VMEM((_EC,), jnp.float32),
            pltpu.VMEM((_EC, _EDIM), jnp.float32),
            pltpu.SemaphoreType.DMA,
        ],
    )
    return kfn(xs, ys, zs, dist1d)


def _sc_body(tpw, e_hbm, s_hbm, r_hbm, c_hbm, sidx, ridx, srow, rrow, outb, sem):
    wid = lax.axis_index("s") * _NC + lax.axis_index("c")
    wbase = wid * tpw
    ncp = (tpw + _CHUNK - 1) // _CHUNK
    last_off = tpw - _CHUNK

    def step(j, carry):
        base = wbase + jnp.minimum(j * _CHUNK, last_off)
        cp_s = pltpu.async_copy(s_hbm.at[pl.ds(base, _CHUNK)], sidx, sem)
        cp_r = pltpu.async_copy(r_hbm.at[pl.ds(base, _CHUNK)], ridx, sem)
        cp_s.wait()
        cp_r.wait()
        cp1 = pltpu.async_copy(e_hbm.at[sidx], srow, sem)
        cp2 = pltpu.async_copy(e_hbm.at[ridx], rrow, sem)
        cp1.wait()
        cp2.wait()
        for g in range(_CHUNK // 16):
            t = lax.iota(jnp.int32, 16) + g * 16

            def col(rowref, c):
                return plsc.load_gather(rowref, [t, jnp.full((16,), c, jnp.int32)])

            sx, sy, sz, sd = col(srow, 0), col(srow, 1), col(srow, 2), col(srow, 3)
            rx, ry, rz = col(rrow, 0), col(rrow, 1), col(rrow, 2)
            cosang = sx * rx + sy * ry + sz * rz
            row = lax.shift_right_logical(t, 1)
            colb = lax.mul(lax.bitwise_and(t, 1), 2)
            plsc.store_scatter(outb, [row, colb], cosang)
            plsc.store_scatter(outb, [row, colb + 1], sd)
        pltpu.sync_copy(outb, c_hbm.at[pl.ds(lax.shift_right_logical(base, 1), _CHUNK // 2)])
        return carry

    lax.fori_loop(0, ncp, step, 0)


def _sc_gather(etab, senders, receivers):
    t = senders.shape[0]
    tpw = t // _NW
    mesh = plsc.VectorSubcoreMesh(
        core_axis_name="c", subcore_axis_name="s", num_cores=_NC, num_subcores=_NS
    )
    kfn = pl.kernel(
        functools.partial(_sc_body, tpw),
        out_type=jax.ShapeDtypeStruct((t // 2, 4), jnp.float32),
        mesh=mesh,
        compiler_params=pltpu.CompilerParams(
            use_tc_tiling_on_sc=False, needs_layout_passes=False
        ),
        scratch_types=[
            pltpu.VMEM((_CHUNK,), jnp.int32),
            pltpu.VMEM((_CHUNK,), jnp.int32),
            pltpu.VMEM((_CHUNK, _EDIM), jnp.float32),
            pltpu.VMEM((_CHUNK, _EDIM), jnp.float32),
            pltpu.VMEM((_CHUNK // 2, 4), jnp.float32),
            pltpu.SemaphoreType.DMA,
        ],
    )
    return kfn(etab, senders, receivers)


_PK = 2
_PLANES = _PK * _NF
_CONSTS = np.zeros((8, _PLANES), np.float32)
_CONSTS[0] = np.tile(_ZF[0, :_NF], _PK)
_CONSTS[1] = np.tile(_ORDF[0, :_NF], _PK)
_CONSTS[2] = np.tile(_NRMF[0, :_NF], _PK)


def _basis_body(k_ref, c_ref, o_ref):
    cb = c_ref[...]
    bs = cb.shape[0]
    cosang = jnp.concatenate(
        [
            lax.broadcast_in_dim(cb[:, 2 * p : 2 * p + 1], (bs, _NF), (0, 1))
            for p in range(_PK)
        ],
        axis=1,
    )
    d = jnp.concatenate(
        [
            lax.broadcast_in_dim(cb[:, 2 * p + 1 : 2 * p + 2], (bs, _NF), (0, 1))
            for p in range(_PK)
        ],
        axis=1,
    )
    zf = k_ref[0:1, :]
    ordf = k_ref[1:2, :]
    nrmf = k_ref[2:3, :]

    x = d * zf
    sx = jnp.sin(x)
    cx = jnp.cos(x)
    j0 = sx / x
    j1 = sx / (x * x) - cx / x
    jsel = jnp.where(ordf >= 1.0, j1, j0)
    jm1, jc = j0, j1
    for l in range(1, NSPH - 1):
        jn = (2 * l + 1) / x * jc - jm1
        jm1, jc = jc, jn
        jsel = jnp.where(ordf >= float(l + 1), jn, jsel)

    p_prev = jnp.ones_like(cosang)
    p_cur = cosang
    csel = jnp.where(ordf >= 1.0, _SPHC[1] * p_cur, _SPHC[0] * p_prev)
    for l in range(1, NSPH - 1):
        p_next = ((2 * l + 1) * cosang * p_cur - l * p_prev) * (1.0 / (l + 1))
        p_prev, p_cur = p_cur, p_next
        csel = jnp.where(ordf >= float(l + 1), _SPHC[l + 1] * p_cur, csel)

    a = -(ENV_P + 1) * (ENV_P + 2) / 2.0
    b = float(ENV_P * (ENV_P + 2))
    c = -ENV_P * (ENV_P + 1) / 2.0
    env = 1.0 / d + a * d ** (ENV_P - 1) + b * d ** ENV_P + c * d ** (ENV_P + 1)
    env = jnp.where(d < 1.0, env, jnp.zeros_like(env))

    out = jsel * nrmf * csel * env
    o_ref[...] = out


def _basis(cstream, t, bs=1600):
    rows = t // _PK
    cpk = cstream
    out = pl.pallas_call(
        _basis_body,
        grid=(rows // bs,),
        in_specs=[
            pl.BlockSpec((8, _PLANES), lambda i: (0, 0)),
            pl.BlockSpec((bs, 2 * _PK), lambda i: (i, 0)),
        ],
        out_specs=pl.BlockSpec((bs, _PLANES), lambda i: (i, 0)),
        out_shape=jax.ShapeDtypeStruct((rows, _PLANES), jnp.float32),
    )(jnp.asarray(_CONSTS), cpk)
    return out.reshape(t, _NF)


def kernel(distances, senders, receivers):
    t = senders.shape[0]
    dist = jnp.linalg.norm(distances, axis=-1)
    etab = _edge_table(distances, dist)
    cstream = _sc_gather(etab, senders.astype(jnp.int32), receivers.astype(jnp.int32))
    return _basis(cstream, t)


# speedup vs baseline: 1.9282x; 1.0558x over previous
import functools

import jax
import jax.numpy as jnp
import numpy as np
from jax import lax
from jax.experimental import pallas as pl
from jax.experimental.pallas import tpu as pltpu
from jax.experimental.pallas import tpu_sc as plsc

NSPH = 7
NRAD = 6
CUT = 5.0
ENV_P = 6


def _sph_jn_np(x, n):
    x = np.asarray(x, dtype=np.float64)
    j0 = np.sin(x) / x
    if n == 0:
        return j0
    j1 = np.sin(x) / x ** 2 - np.cos(x) / x
    jm1, jc = j0, j1
    for l in range(1, n):
        jm1, jc = jc, (2 * l + 1) / x * jc - jm1
    return jc


def _sph_jn_zeros(n, k):
    zerosj = np.zeros((n, k), dtype=np.float64)
    zerosj[0] = np.arange(1, k + 1) * np.pi
    points = np.arange(1, k + n) * np.pi
    for i in range(1, n):
        m = k + n - 1 - i
        racines = np.zeros(m)
        for j in range(m):
            a, b = points[j], points[j + 1]
            fa = _sph_jn_np(a, i)
            for _ in range(100):
                c = 0.5 * (a + b)
                fc = _sph_jn_np(c, i)
                if fa * fc <= 0:
                    b = c
                else:
                    a, fa = c, fc
            racines[j] = 0.5 * (a + b)
        points = racines
        zerosj[i, :k] = racines[:k]
    return zerosj


_BZ = _sph_jn_zeros(NSPH, NRAD)
_BN = np.zeros((NSPH, NRAD))
for _o in range(NSPH):
    _BN[_o] = 1.0 / np.sqrt(0.5 * _sph_jn_np(_BZ[_o], _o + 1) ** 2)

_NF = NSPH * NRAD
_LANES = 48

_ZF = np.ones((1, _LANES), np.float32)
_ZF[0, :_NF] = _BZ.reshape(-1).astype(np.float32)
_NRMF = np.zeros((1, _LANES), np.float32)
_NRMF[0, :_NF] = _BN.reshape(-1).astype(np.float32)
_ORDF = np.zeros((1, _LANES), np.float32)
_ORDF[0, :_NF] = np.repeat(np.arange(NSPH), NRAD).astype(np.float32)
_SPHC = [np.sqrt((2 * l + 1) / (4 * np.pi)).astype(np.float32) for l in range(NSPH)]

_NC = 2
_NS = 16
_NW = _NC * _NS
_CHUNK = 128
_EDIM = 8


_EC = 1024


def _sc_edge_body(npw, xs, ys, zs, ds, e_hbm, xb, yb, zb, db, eb, sem):
    wid = lax.axis_index("s") * _NC + lax.axis_index("c")
    wbase = wid * npw
    ncp = (npw + _EC - 1) // _EC
    last_off = npw - _EC

    def step(j, carry):
        base = wbase + jnp.minimum(j * _EC, last_off)
        cps = [
            pltpu.async_copy(src.at[pl.ds(base, _EC)], dst, sem)
            for src, dst in ((xs, xb), (ys, yb), (zs, zb), (ds, db))
        ]
        for cp in cps:
            cp.wait()
        for g in range(_EC // 16):
            t = lax.iota(jnp.int32, 16) + g * 16
            sl = pl.ds(g * 16, 16)
            dv = db[sl]
            nx = xb[sl] / dv
            ny = yb[sl] / dv
            nz = zb[sl] / dv
            dd = dv / CUT
            for c, val in ((0, nx), (1, ny), (2, nz), (3, dd)):
                plsc.store_scatter(eb, [t, jnp.full((16,), c, jnp.int32)], val)
        pltpu.sync_copy(eb, e_hbm.at[pl.ds(base, _EC)])
        return carry

    lax.fori_loop(0, ncp, step, 0)


def _edge_table(distances, dist1d):
    n = distances.shape[0]
    npw = n // _NW
    xs = distances[:, 0]
    ys = distances[:, 1]
    zs = distances[:, 2]
    mesh = plsc.VectorSubcoreMesh(
        core_axis_name="c", subcore_axis_name="s", num_cores=_NC, num_subcores=_NS
    )
    kfn = pl.kernel(
        functools.partial(_sc_edge_body, npw),
        out_type=jax.ShapeDtypeStruct((n, _EDIM), jnp.float32),
        mesh=mesh,
        compiler_params=pltpu.CompilerParams(
            use_tc_tiling_on_sc=False, needs_layout_passes=False
        ),
        scratch_types=[
            pltpu.VMEM((_EC,), jnp.float32),
            pltpu.VMEM((_EC,), jnp.float32),
            pltpu.VMEM((_EC,), jnp.float32),
            pltpu.VMEM((_EC,), jnp.float32),
            pltpu.VMEM((_EC, _EDIM), jnp.float32),
            pltpu.SemaphoreType.DMA,
        ],
    )
    return kfn(xs, ys, zs, dist1d)


def _sc_body(tpw, e_hbm, s_hbm, r_hbm, c_hbm, *bufs):
    (
        sidx0, ridx0, srow0, rrow0, outb0, semi0, sema0, semo0,
        sidx1, ridx1, srow1, rrow1, outb1, semi1, sema1, semo1,
    ) = bufs
    sidx = (sidx0, sidx1)
    ridx = (ridx0, ridx1)
    srow = (srow0, srow1)
    rrow = (rrow0, rrow1)
    outb = (outb0, outb1)
    semi = (semi0, semi1)
    sema = (sema0, sema1)
    semo = (semo0, semo1)

    wid = lax.axis_index("s") * _NC + lax.axis_index("c")
    wbase = wid * tpw
    ncp = (tpw + _CHUNK - 1) // _CHUNK
    last_off = tpw - _CHUNK

    def base(j):
        return wbase + jnp.minimum(j * _CHUNK, last_off)

    def issue_idx(j, b):
        pltpu.async_copy(s_hbm.at[pl.ds(base(j), _CHUNK)], sidx[b], semi[b])
        pltpu.async_copy(r_hbm.at[pl.ds(base(j), _CHUNK)], ridx[b], semi[b])

    def wait_idx(j, b):
        pltpu.make_async_copy(s_hbm.at[pl.ds(base(j), _CHUNK)], sidx[b], semi[b]).wait()
        pltpu.make_async_copy(r_hbm.at[pl.ds(base(j), _CHUNK)], ridx[b], semi[b]).wait()

    def issue_gather(b):
        pltpu.async_copy(e_hbm.at[sidx[b]], srow[b], sema[b])
        pltpu.async_copy(e_hbm.at[ridx[b]], rrow[b], sema[b])

    def wait_gather(b):
        pltpu.make_async_copy(e_hbm.at[sidx[b]], srow[b], sema[b]).wait()
        pltpu.make_async_copy(e_hbm.at[ridx[b]], rrow[b], sema[b]).wait()

    def out_slice(j):
        return c_hbm.at[pl.ds(lax.shift_right_logical(base(j), 1), _CHUNK // 2)]

    def compute(b):
        for g in range(_CHUNK // 16):
            t = lax.iota(jnp.int32, 16) + g * 16

            def col(rowref, c):
                return plsc.load_gather(rowref, [t, jnp.full((16,), c, jnp.int32)])

            sx, sy, sz, sd = (
                col(srow[b], 0), col(srow[b], 1), col(srow[b], 2), col(srow[b], 3)
            )
            rx, ry, rz = col(rrow[b], 0), col(rrow[b], 1), col(rrow[b], 2)
            cosang = sx * rx + sy * ry + sz * rz
            row = lax.shift_right_logical(t, 1)
            colb = lax.mul(lax.bitwise_and(t, 1), 2)
            plsc.store_scatter(outb[b], [row, colb], cosang)
            plsc.store_scatter(outb[b], [row, colb + 1], sd)

    issue_idx(jnp.int32(0), 0)
    issue_idx(jnp.int32(1), 1)
    wait_idx(jnp.int32(0), 0)
    issue_gather(0)

    def half(j, b):
        wait_gather(b)

        @pl.when(j + 2 < ncp)
        def _():
            issue_idx(j + 2, b)

        @pl.when(j + 1 < ncp)
        def _():
            wait_idx(j + 1, 1 - b)
            issue_gather(1 - b)

        @pl.when(j >= 2)
        def _():
            pltpu.make_async_copy(outb[b], out_slice(j - 2), semo[b]).wait()

        compute(b)
        pltpu.async_copy(outb[b], out_slice(j), semo[b])

    def step(i, carry):
        half(2 * i, 0)
        half(2 * i + 1, 1)
        return carry

    lax.fori_loop(0, ncp // 2, step, 0)
    pltpu.make_async_copy(outb[0], out_slice(jnp.int32(ncp - 2)), semo[0]).wait()
    pltpu.make_async_copy(outb[1], out_slice(jnp.int32(ncp - 1)), semo[1]).wait()


def _sc_gather(etab, senders, receivers):
    t = senders.shape[0]
    tpw = t // _NW
    mesh = plsc.VectorSubcoreMesh(
        core_axis_name="c", subcore_axis_name="s", num_cores=_NC, num_subcores=_NS
    )
    bufset = [
        pltpu.VMEM((_CHUNK,), jnp.int32),
        pltpu.VMEM((_CHUNK,), jnp.int32),
        pltpu.VMEM((_CHUNK, _EDIM), jnp.float32),
        pltpu.VMEM((_CHUNK, _EDIM), jnp.float32),
        pltpu.VMEM((_CHUNK // 2, 4), jnp.float32),
        pltpu.SemaphoreType.DMA,
        pltpu.SemaphoreType.DMA,
        pltpu.SemaphoreType.DMA,
    ]
    kfn = pl.kernel(
        functools.partial(_sc_body, tpw),
        out_type=jax.ShapeDtypeStruct((t // 2, 4), jnp.float32),
        mesh=mesh,
        compiler_params=pltpu.CompilerParams(
            use_tc_tiling_on_sc=False, needs_layout_passes=False
        ),
        scratch_types=bufset + bufset,
    )
    return kfn(etab, senders, receivers)


_PK = 2
_PLANES = _PK * _NF
_CONSTS = np.zeros((8, _PLANES), np.float32)
_CONSTS[0] = np.tile(_ZF[0, :_NF], _PK)
_CONSTS[1] = np.tile(_ORDF[0, :_NF], _PK)
_CONSTS[2] = np.tile(_NRMF[0, :_NF], _PK)


def _basis_body(k_ref, c_ref, o_ref):
    cb = c_ref[...]
    bs = cb.shape[0]
    cosang = jnp.concatenate(
        [
            lax.broadcast_in_dim(cb[:, 2 * p : 2 * p + 1], (bs, _NF), (0, 1))
            for p in range(_PK)
        ],
        axis=1,
    )
    d = jnp.concatenate(
        [
            lax.broadcast_in_dim(cb[:, 2 * p + 1 : 2 * p + 2], (bs, _NF), (0, 1))
            for p in range(_PK)
        ],
        axis=1,
    )
    zf = k_ref[0:1, :]
    ordf = k_ref[1:2, :]
    nrmf = k_ref[2:3, :]

    x = d * zf
    sx = jnp.sin(x)
    cx = jnp.cos(x)
    j0 = sx / x
    j1 = sx / (x * x) - cx / x
    jsel = jnp.where(ordf >= 1.0, j1, j0)
    jm1, jc = j0, j1
    for l in range(1, NSPH - 1):
        jn = (2 * l + 1) / x * jc - jm1
        jm1, jc = jc, jn
        jsel = jnp.where(ordf >= float(l + 1), jn, jsel)

    p_prev = jnp.ones_like(cosang)
    p_cur = cosang
    csel = jnp.where(ordf >= 1.0, _SPHC[1] * p_cur, _SPHC[0] * p_prev)
    for l in range(1, NSPH - 1):
        p_next = ((2 * l + 1) * cosang * p_cur - l * p_prev) * (1.0 / (l + 1))
        p_prev, p_cur = p_cur, p_next
        csel = jnp.where(ordf >= float(l + 1), _SPHC[l + 1] * p_cur, csel)

    a = -(ENV_P + 1) * (ENV_P + 2) / 2.0
    b = float(ENV_P * (ENV_P + 2))
    c = -ENV_P * (ENV_P + 1) / 2.0
    env = 1.0 / d + a * d ** (ENV_P - 1) + b * d ** ENV_P + c * d ** (ENV_P + 1)
    env = jnp.where(d < 1.0, env, jnp.zeros_like(env))

    out = jsel * nrmf * csel * env
    o_ref[...] = out


def _basis(cstream, t, bs=1600):
    rows = t // _PK
    cpk = cstream
    out = pl.pallas_call(
        _basis_body,
        grid=(rows // bs,),
        in_specs=[
            pl.BlockSpec((8, _PLANES), lambda i: (0, 0)),
            pl.BlockSpec((bs, 2 * _PK), lambda i: (i, 0)),
        ],
        out_specs=pl.BlockSpec((bs, _PLANES), lambda i: (i, 0)),
        out_shape=jax.ShapeDtypeStruct((rows, _PLANES), jnp.float32),
    )(jnp.asarray(_CONSTS), cpk)
    return out.reshape(t, _NF)


def kernel(distances, senders, receivers):
    t = senders.shape[0]
    dist = jnp.linalg.norm(distances, axis=-1)
    etab = _edge_table(distances, dist)
    cstream = _sc_gather(etab, senders.astype(jnp.int32), receivers.astype(jnp.int32))
    return _basis(cstream, t)


# speedup vs baseline: 1.9555x; 1.0142x over previous
import functools

import jax
import jax.numpy as jnp
import numpy as np
from jax import lax
from jax.experimental import pallas as pl
from jax.experimental.pallas import tpu as pltpu
from jax.experimental.pallas import tpu_sc as plsc

NSPH = 7
NRAD = 6
CUT = 5.0
ENV_P = 6


def _sph_jn_np(x, n):
    x = np.asarray(x, dtype=np.float64)
    j0 = np.sin(x) / x
    if n == 0:
        return j0
    j1 = np.sin(x) / x ** 2 - np.cos(x) / x
    jm1, jc = j0, j1
    for l in range(1, n):
        jm1, jc = jc, (2 * l + 1) / x * jc - jm1
    return jc


def _sph_jn_zeros(n, k):
    zerosj = np.zeros((n, k), dtype=np.float64)
    zerosj[0] = np.arange(1, k + 1) * np.pi
    points = np.arange(1, k + n) * np.pi
    for i in range(1, n):
        m = k + n - 1 - i
        racines = np.zeros(m)
        for j in range(m):
            a, b = points[j], points[j + 1]
            fa = _sph_jn_np(a, i)
            for _ in range(100):
                c = 0.5 * (a + b)
                fc = _sph_jn_np(c, i)
                if fa * fc <= 0:
                    b = c
                else:
                    a, fa = c, fc
            racines[j] = 0.5 * (a + b)
        points = racines
        zerosj[i, :k] = racines[:k]
    return zerosj


_BZ = _sph_jn_zeros(NSPH, NRAD)
_BN = np.zeros((NSPH, NRAD))
for _o in range(NSPH):
    _BN[_o] = 1.0 / np.sqrt(0.5 * _sph_jn_np(_BZ[_o], _o + 1) ** 2)

_NF = NSPH * NRAD
_LANES = 48

_ZF = np.ones((1, _LANES), np.float32)
_ZF[0, :_NF] = _BZ.reshape(-1).astype(np.float32)
_NRMF = np.zeros((1, _LANES), np.float32)
_NRMF[0, :_NF] = _BN.reshape(-1).astype(np.float32)
_ORDF = np.zeros((1, _LANES), np.float32)
_ORDF[0, :_NF] = np.repeat(np.arange(NSPH), NRAD).astype(np.float32)
_SPHC = [np.sqrt((2 * l + 1) / (4 * np.pi)).astype(np.float32) for l in range(NSPH)]

_NC = 2
_NS = 16
_NW = _NC * _NS
_CHUNK = 128
_EDIM = 8


_EC = 1024


def _sc_edge_body(npw, xs, ys, zs, ds, e_hbm, xb, yb, zb, db, eb, sem):
    wid = lax.axis_index("s") * _NC + lax.axis_index("c")
    wbase = wid * npw
    ncp = (npw + _EC - 1) // _EC
    last_off = npw - _EC

    def step(j, carry):
        base = wbase + jnp.minimum(j * _EC, last_off)
        cps = [
            pltpu.async_copy(src.at[pl.ds(base, _EC)], dst, sem)
            for src, dst in ((xs, xb), (ys, yb), (zs, zb), (ds, db))
        ]
        for cp in cps:
            cp.wait()
        for g in range(_EC // 16):
            t = lax.iota(jnp.int32, 16) + g * 16
            sl = pl.ds(g * 16, 16)
            dv = db[sl]
            nx = xb[sl] / dv
            ny = yb[sl] / dv
            nz = zb[sl] / dv
            dd = dv / CUT
            for c, val in ((0, nx), (1, ny), (2, nz), (3, dd)):
                plsc.store_scatter(eb, [t, jnp.full((16,), c, jnp.int32)], val)
        pltpu.sync_copy(eb, e_hbm.at[pl.ds(base, _EC)])
        return carry

    lax.fori_loop(0, ncp, step, 0)


def _edge_table(distances, dist1d):
    n = distances.shape[0]
    npw = n // _NW
    xs = distances[:, 0]
    ys = distances[:, 1]
    zs = distances[:, 2]
    mesh = plsc.VectorSubcoreMesh(
        core_axis_name="c", subcore_axis_name="s", num_cores=_NC, num_subcores=_NS
    )
    kfn = pl.kernel(
        functools.partial(_sc_edge_body, npw),
        out_type=jax.ShapeDtypeStruct((n, _EDIM), jnp.float32),
        mesh=mesh,
        compiler_params=pltpu.CompilerParams(
            use_tc_tiling_on_sc=False, needs_layout_passes=False
        ),
        scratch_types=[
            pltpu.VMEM((_EC,), jnp.float32),
            pltpu.VMEM((_EC,), jnp.float32),
            pltpu.VMEM((_EC,), jnp.float32),
            pltpu.VMEM((_EC,), jnp.float32),
            pltpu.VMEM((_EC, _EDIM), jnp.float32),
            pltpu.SemaphoreType.DMA,
        ],
    )
    return kfn(xs, ys, zs, dist1d)


def _sc_body(tpw, e_hbm, s_hbm, r_hbm, c_hbm, *bufs):
    (
        sidx0, ridx0, srow0, rrow0, outb0, semi0, sema0, semo0,
        sidx1, ridx1, srow1, rrow1, outb1, semi1, sema1, semo1,
    ) = bufs
    sidx = (sidx0, sidx1)
    ridx = (ridx0, ridx1)
    srow = (srow0, srow1)
    rrow = (rrow0, rrow1)
    outb = (outb0, outb1)
    semi = (semi0, semi1)
    sema = (sema0, sema1)
    semo = (semo0, semo1)

    wid = lax.axis_index("s") * _NC + lax.axis_index("c")
    wbase = wid * tpw
    ncp = (tpw + _CHUNK - 1) // _CHUNK
    last_off = tpw - _CHUNK

    def base(j):
        return wbase + jnp.minimum(j * _CHUNK, last_off)

    def issue_idx(j, b):
        pltpu.async_copy(s_hbm.at[pl.ds(base(j), _CHUNK)], sidx[b], semi[b])
        pltpu.async_copy(r_hbm.at[pl.ds(base(j), _CHUNK)], ridx[b], semi[b])

    def wait_idx(j, b):
        pltpu.make_async_copy(s_hbm.at[pl.ds(base(j), _CHUNK)], sidx[b], semi[b]).wait()
        pltpu.make_async_copy(r_hbm.at[pl.ds(base(j), _CHUNK)], ridx[b], semi[b]).wait()

    def issue_gather(b):
        pltpu.async_copy(e_hbm.at[sidx[b]], srow[b], sema[b])
        pltpu.async_copy(e_hbm.at[ridx[b]], rrow[b], sema[b])

    def wait_gather(b):
        pltpu.make_async_copy(e_hbm.at[sidx[b]], srow[b], sema[b]).wait()
        pltpu.make_async_copy(e_hbm.at[ridx[b]], rrow[b], sema[b]).wait()

    def out_slice(j):
        return c_hbm.at[pl.ds(lax.shift_right_logical(base(j), 1), _CHUNK // 2)]

    def compute(b):
        for g in range(_CHUNK // 16):
            t = lax.iota(jnp.int32, 16) + g * 16

            def col(rowref, c):
                return plsc.load_gather(rowref, [t, jnp.full((16,), c, jnp.int32)])

            sx, sy, sz, sd = (
                col(srow[b], 0), col(srow[b], 1), col(srow[b], 2), col(srow[b], 3)
            )
            rx, ry, rz = col(rrow[b], 0), col(rrow[b], 1), col(rrow[b], 2)
            cosang = sx * rx + sy * ry + sz * rz
            row = lax.shift_right_logical(t, 1)
            colb = lax.mul(lax.bitwise_and(t, 1), 2)
            plsc.store_scatter(outb[b], [row, colb], cosang)
            plsc.store_scatter(outb[b], [row, colb + 1], sd)

    issue_idx(jnp.int32(0), 0)
    issue_idx(jnp.int32(1), 1)
    wait_idx(jnp.int32(0), 0)
    issue_gather(0)

    def half(j, b):
        wait_gather(b)

        @pl.when(j + 2 < ncp)
        def _():
            issue_idx(j + 2, b)

        @pl.when(j + 1 < ncp)
        def _():
            wait_idx(j + 1, 1 - b)
            issue_gather(1 - b)

        @pl.when(j >= 2)
        def _():
            pltpu.make_async_copy(outb[b], out_slice(j - 2), semo[b]).wait()

        compute(b)
        pltpu.async_copy(outb[b], out_slice(j), semo[b])

    def step(i, carry):
        half(2 * i, 0)
        half(2 * i + 1, 1)
        return carry

    lax.fori_loop(0, ncp // 2, step, 0)
    pltpu.make_async_copy(outb[0], out_slice(jnp.int32(ncp - 2)), semo[0]).wait()
    pltpu.make_async_copy(outb[1], out_slice(jnp.int32(ncp - 1)), semo[1]).wait()


def _sc_gather(etab, senders, receivers):
    t = senders.shape[0]
    tpw = t // _NW
    mesh = plsc.VectorSubcoreMesh(
        core_axis_name="c", subcore_axis_name="s", num_cores=_NC, num_subcores=_NS
    )
    bufset = [
        pltpu.VMEM((_CHUNK,), jnp.int32),
        pltpu.VMEM((_CHUNK,), jnp.int32),
        pltpu.VMEM((_CHUNK, _EDIM), jnp.float32),
        pltpu.VMEM((_CHUNK, _EDIM), jnp.float32),
        pltpu.VMEM((_CHUNK // 2, 4), jnp.float32),
        pltpu.SemaphoreType.DMA,
        pltpu.SemaphoreType.DMA,
        pltpu.SemaphoreType.DMA,
    ]
    kfn = pl.kernel(
        functools.partial(_sc_body, tpw),
        out_type=jax.ShapeDtypeStruct((t // 2, 4), jnp.float32),
        mesh=mesh,
        compiler_params=pltpu.CompilerParams(
            use_tc_tiling_on_sc=False, needs_layout_passes=False
        ),
        scratch_types=bufset + bufset,
    )
    return kfn(etab, senders, receivers)


_PK = 2
_PLANES = _PK * _NF
_CONSTS = np.zeros((8, _PLANES), np.float32)
_CONSTS[0] = np.tile(_ZF[0, :_NF], _PK)
_CONSTS[1] = np.tile(_ORDF[0, :_NF], _PK)
_CONSTS[2] = np.tile(_NRMF[0, :_NF], _PK)


def _basis_body(k_ref, c_ref, o_ref):
    cb = c_ref[...]
    bs = cb.shape[0]
    cosang = jnp.concatenate(
        [
            lax.broadcast_in_dim(cb[:, 2 * p : 2 * p + 1], (bs, _NF), (0, 1))
            for p in range(_PK)
        ],
        axis=1,
    )
    d = jnp.concatenate(
        [
            lax.broadcast_in_dim(cb[:, 2 * p + 1 : 2 * p + 2], (bs, _NF), (0, 1))
            for p in range(_PK)
        ],
        axis=1,
    )
    zf = k_ref[0:1, :]
    ordf = k_ref[1:2, :]
    nrmf = k_ref[2:3, :]

    x = d * zf
    sx = jnp.sin(x)
    cx = jnp.cos(x)
    j0 = sx / x
    j1 = sx / (x * x) - cx / x
    jsel = jnp.where(ordf >= 1.0, j1, j0)
    jm1, jc = j0, j1
    for l in range(1, NSPH - 1):
        jn = (2 * l + 1) / x * jc - jm1
        jm1, jc = jc, jn
        jsel = jnp.where(ordf >= float(l + 1), jn, jsel)

    p_prev = jnp.ones_like(cosang)
    p_cur = cosang
    csel = jnp.where(ordf >= 1.0, _SPHC[1] * p_cur, _SPHC[0] * p_prev)
    for l in range(1, NSPH - 1):
        p_next = ((2 * l + 1) * cosang * p_cur - l * p_prev) * (1.0 / (l + 1))
        p_prev, p_cur = p_cur, p_next
        csel = jnp.where(ordf >= float(l + 1), _SPHC[l + 1] * p_cur, csel)

    a = -(ENV_P + 1) * (ENV_P + 2) / 2.0
    b = float(ENV_P * (ENV_P + 2))
    c = -ENV_P * (ENV_P + 1) / 2.0
    env = 1.0 / d + a * d ** (ENV_P - 1) + b * d ** ENV_P + c * d ** (ENV_P + 1)
    env = jnp.where(d < 1.0, env, jnp.zeros_like(env))

    out = jsel * nrmf * csel * env
    o_ref[...] = out


def _basis(cstream, t, bs=1600):
    rows = t // _PK
    cpk = cstream
    out = pl.pallas_call(
        _basis_body,
        grid=(rows // bs,),
        in_specs=[
            pl.BlockSpec((8, _PLANES), lambda i: (0, 0)),
            pl.BlockSpec((bs, 2 * _PK), lambda i: (i, 0)),
        ],
        out_specs=pl.BlockSpec((bs, _PLANES), lambda i: (i, 0)),
        out_shape=jax.ShapeDtypeStruct((rows, _PLANES), jnp.float32),
    )(jnp.asarray(_CONSTS), cpk)
    return out.reshape(t, _NF)


def _basis_sliced(c1, c2, t1, t, bs=1600):
    rows = t // _PK
    nb1 = (t1 // _PK) // bs
    nb2 = (rows - t1 // _PK) // bs
    consts = jnp.asarray(_CONSTS)
    cspec = pl.BlockSpec((8, _PLANES), lambda i: (0, 0))
    out1 = pl.pallas_call(
        _basis_body,
        grid=(nb1,),
        in_specs=[cspec, pl.BlockSpec((bs, 2 * _PK), lambda i: (i, 0))],
        out_specs=pl.BlockSpec((bs, _PLANES), lambda i: (i, 0)),
        out_shape=jax.ShapeDtypeStruct((rows, _PLANES), jnp.float32),
    )(consts, c1)

    def body2(k_ref, c_ref, prev_ref, o_ref):
        del prev_ref
        _basis_body(k_ref, c_ref, o_ref)

    out = pl.pallas_call(
        body2,
        grid=(nb2,),
        in_specs=[
            cspec,
            pl.BlockSpec((bs, 2 * _PK), lambda i: (i, 0)),
            pl.BlockSpec(memory_space=pl.ANY),
        ],
        out_specs=pl.BlockSpec((bs, _PLANES), lambda i: (i + nb1, 0)),
        out_shape=jax.ShapeDtypeStruct((rows, _PLANES), jnp.float32),
        input_output_aliases={2: 0},
    )(consts, c2, out1)
    return out.reshape(t, _NF)


def kernel(distances, senders, receivers):
    t = senders.shape[0]
    dist = jnp.linalg.norm(distances, axis=-1)
    etab = _edge_table(distances, dist)
    senders = senders.astype(jnp.int32)
    receivers = receivers.astype(jnp.int32)
    t1 = 409600 if t == 800000 else t
    if t1 == t:
        cstream = _sc_gather(etab, senders, receivers)
        return _basis(cstream, t)
    c1 = _sc_gather(etab, senders[:t1], receivers[:t1])
    c2 = _sc_gather(etab, senders[t1:], receivers[t1:])
    return _basis_sliced(c1, c2, t1, t)
